# baseline (speedup 1.0000x reference)
import functools

import jax
import jax.numpy as jnp
from jax.experimental import pallas as pl
from jax.experimental.pallas import tpu as pltpu

_EPS = 1e-10
_TOP_P = 0.5


def _body(x_ref, masks_ref, w_ref, out_ref, loss_ref, imp_acc, ent_acc):
    i = pl.program_id(0)
    nsteps = pl.num_programs(0)

    @pl.when(i == 0)
    def _init():
        imp_acc[...] = jnp.zeros_like(imp_acc)
        ent_acc[0] = jnp.float32(0.0)

    xb = x_ref[...]
    K, L, _ = xb.shape
    w = w_ref[...]

    m0 = masks_ref[0]
    m1 = masks_ref[1]
    m2 = masks_ref[2]
    rows = jax.lax.broadcasted_iota(jnp.int32, (L, L), 0)
    cols = jax.lax.broadcasted_iota(jnp.int32, (L, L), 1)
    eye = (rows == cols).astype(jnp.float32)

    for k in range(K):
        lg = jnp.dot(xb[k], w, preferred_element_type=jnp.float32)
        c0 = lg[:, 0:1]
        c1 = lg[:, 1:2]
        c2 = lg[:, 2:3]

        cm = jnp.maximum(jnp.maximum(c0, c1), c2)
        e0 = jnp.exp(c0 - cm)
        e1 = jnp.exp(c1 - cm)
        e2 = jnp.exp(c2 - cm)
        s = e0 + e1 + e2
        p0 = e0 / s
        p1 = e1 / s
        p2 = e2 / s

        ge01 = p0 >= p1
        ge02 = p0 >= p2
        ge12 = p1 >= p2
        is0 = ge01 & ge02
        is1 = (~is0) & ge12
        is2 = ~(is0 | is1)
        pmax = jnp.where(is0, p0, jnp.where(is1, p1, p2))
        sec0 = (is1 & ge02) | (is2 & ge01)
        sec1 = (is0 & ge12) | (is2 & ~ge01)
        sec2 = (is0 & ~ge12) | (is1 & ~ge02)
        psec = jnp.where(sec0, p0, jnp.where(sec1, p1, p2))

        keep2 = pmax <= _TOP_P
        k2f = keep2.astype(jnp.float32)
        g0 = is0.astype(jnp.float32) + k2f * sec0.astype(jnp.float32)
        g1 = is1.astype(jnp.float32) + k2f * sec1.astype(jnp.float32)
        g2 = is2.astype(jnp.float32) + k2f * sec2.astype(jnp.float32)

        ent = -(p0 * jnp.log(p0 + _EPS)
                + p1 * jnp.log(p1 + _EPS)
                + p2 * jnp.log(p2 + _EPS))
        ent_acc[0] += jnp.sum(ent)

        imp_acc[:, 0:1] += pmax
        imp_acc[:, 1:2] += psec * k2f

        out_ref[k] = g0 * m0 + g1 * m1 + g2 * m2 + eye

    @pl.when(i == nsteps - 1)
    def _finalize():
        imp = imp_acc[...]
        n = jnp.float32(3 * L)
        mean = jnp.sum(imp) / n
        d = imp - mean
        ssq = jnp.sum(d * d) + jnp.float32(L) * mean * mean
        var = ssq / (n - 1.0)
        loss_imp = var / (mean * mean + _EPS)
        loss_dyn = ent_acc[0] / jnp.float32(3 * 512)
        loss_ref[0] = loss_imp + 0.1 * loss_dyn


@functools.partial(jax.jit, static_argnames=())
def kernel(x, masks, W_gate):
    B, H, L, _ = x.shape
    BH = B * H
    K = 8
    x_flat = x.reshape(BH, L, L)
    masks_t = jnp.transpose(masks, (1, 0, 2))

    out, loss2d = pl.pallas_call(
        _body,
        grid=(BH // K,),
        in_specs=[
            pl.BlockSpec((K, L, L), lambda i: (i, 0, 0)),
            pl.BlockSpec((3, L, L), lambda i: (0, 0, 0)),
            pl.BlockSpec((L, 3), lambda i: (0, 0)),
        ],
        out_specs=[
            pl.BlockSpec((K, L, L), lambda i: (i, 0, 0)),
            pl.BlockSpec(memory_space=pltpu.SMEM),
        ],
        out_shape=[
            jax.ShapeDtypeStruct((BH, L, L), jnp.float32),
            jax.ShapeDtypeStruct((1,), jnp.float32),
        ],
        scratch_shapes=[
            pltpu.VMEM((L, 2), jnp.float32),
            pltpu.SMEM((1,), jnp.float32),
        ],
    )(x_flat, masks_t, W_gate)

    return out.reshape(B, H, L, L), loss2d[0]


# speedup vs baseline: 2.1289x; 2.1289x over previous
import functools

import jax
import jax.numpy as jnp
from jax.experimental import pallas as pl
from jax.experimental.pallas import tpu as pltpu

_EPS = 1e-10
_TOP_P = 0.5


def _body(x_ref, masks_ref, wt_ref, out_ref, loss_ref, imp_acc, ent_acc):
    i = pl.program_id(0)
    nsteps = pl.num_programs(0)

    @pl.when(i == 0)
    def _init():
        imp_acc[...] = jnp.zeros_like(imp_acc)
        ent_acc[0] = jnp.float32(0.0)

    xb = x_ref[...]
    K, L, _ = xb.shape
    N = K * L
    x2d = xb.reshape(N, L)
    wt = wt_ref[...]

    lgT = jax.lax.dot_general(wt, x2d, (((1,), (1,)), ((), ())),
                              preferred_element_type=jnp.float32)
    c0 = lgT[0:1, :]
    c1 = lgT[1:2, :]
    c2 = lgT[2:3, :]

    cm = jnp.maximum(jnp.maximum(c0, c1), c2)
    e0 = jnp.exp(c0 - cm)
    e1 = jnp.exp(c1 - cm)
    e2 = jnp.exp(c2 - cm)
    s = e0 + e1 + e2
    p0 = e0 / s
    p1 = e1 / s
    p2 = e2 / s

    ge01 = p0 >= p1
    ge02 = p0 >= p2
    ge12 = p1 >= p2
    is0 = ge01 & ge02
    is1 = (~is0) & ge12
    is2 = ~(is0 | is1)
    pmax = jnp.where(is0, p0, jnp.where(is1, p1, p2))
    sec0 = (is1 & ge02) | (is2 & ge01)
    sec1 = (is0 & ge12) | (is2 & ~ge01)
    sec2 = (is0 & ~ge12) | (is1 & ~ge02)
    psec = jnp.where(sec0, p0, jnp.where(sec1, p1, p2))

    keep2 = pmax <= _TOP_P
    k2f = keep2.astype(jnp.float32)
    g0 = is0.astype(jnp.float32) + k2f * sec0.astype(jnp.float32)
    g1 = is1.astype(jnp.float32) + k2f * sec1.astype(jnp.float32)
    g2 = is2.astype(jnp.float32) + k2f * sec2.astype(jnp.float32)

    ent = -(p0 * jnp.log(p0 + _EPS)
            + p1 * jnp.log(p1 + _EPS)
            + p2 * jnp.log(p2 + _EPS))
    ent_acc[0] += jnp.sum(ent)

    imp_acc[0:1, :] += pmax
    imp_acc[1:2, :] += psec * k2f

    gpack = jnp.concatenate(
        [g0, g1, g2, jnp.zeros((5, N), jnp.float32)], axis=0)
    gcols = jnp.transpose(gpack)

    m0 = masks_ref[0]
    m1 = masks_ref[1]
    m2 = masks_ref[2]
    rows = jax.lax.broadcasted_iota(jnp.int32, (L, L), 0)
    cols = jax.lax.broadcasted_iota(jnp.int32, (L, L), 1)
    eye = (rows == cols).astype(jnp.float32)

    for k in range(K):
        gk = gcols[k * L:(k + 1) * L, :]
        out_ref[k] = (gk[:, 0:1] * m0 + gk[:, 1:2] * m1
                      + gk[:, 2:3] * m2 + eye)

    @pl.when(i == nsteps - 1)
    def _finalize():
        imp = imp_acc[:, 0:L]
        for k in range(1, K):
            imp = imp + imp_acc[:, k * L:(k + 1) * L]
        n = jnp.float32(3 * L)
        mean = jnp.sum(imp) / n
        d = imp - mean
        ssq = jnp.sum(d * d) + jnp.float32(L) * mean * mean
        var = ssq / (n - 1.0)
        loss_imp = var / (mean * mean + _EPS)
        loss_dyn = ent_acc[0] / jnp.float32(3 * 512)
        loss_ref[0] = loss_imp + 0.1 * loss_dyn


@functools.partial(jax.jit, static_argnames=())
def kernel(x, masks, W_gate):
    B, H, L, _ = x.shape
    BH = B * H
    K = 8
    x_flat = x.reshape(BH, L, L)
    masks_t = jnp.transpose(masks, (1, 0, 2))
    w_t = jnp.transpose(W_gate)

    out, loss1 = pl.pallas_call(
        _body,
        grid=(BH // K,),
        in_specs=[
            pl.BlockSpec((K, L, L), lambda i: (i, 0, 0)),
            pl.BlockSpec((3, L, L), lambda i: (0, 0, 0)),
            pl.BlockSpec((3, L), lambda i: (0, 0)),
        ],
        out_specs=[
            pl.BlockSpec((K, L, L), lambda i: (i, 0, 0)),
            pl.BlockSpec(memory_space=pltpu.SMEM),
        ],
        out_shape=[
            jax.ShapeDtypeStruct((BH, L, L), jnp.float32),
            jax.ShapeDtypeStruct((1,), jnp.float32),
        ],
        scratch_shapes=[
            pltpu.VMEM((2, K * L), jnp.float32),
            pltpu.SMEM((1,), jnp.float32),
        ],
    )(x_flat, masks_t, w_t)

    return out.reshape(B, H, L, L), loss1[0]


# speedup vs baseline: 5.2467x; 2.4645x over previous
import functools

import jax
import jax.numpy as jnp
from jax.experimental import pallas as pl
from jax.experimental.pallas import tpu as pltpu

_EPS = 1e-10
_TOP_P = 0.5


def _body(x_ref, masks_ref, wt_ref, out_ref, loss_ref, mb_ref, imp_acc,
          ent_acc):
    i = pl.program_id(0)
    nsteps = pl.num_programs(0)
    L = x_ref.shape[1]
    H = x_ref.shape[2]
    N = L * H

    @pl.when(i == 0)
    def _init():
        imp_acc[...] = jnp.zeros_like(imp_acc)
        ent_acc[0] = jnp.float32(0.0)
        for e in range(3):
            mb_ref[e] = jnp.broadcast_to(
                masks_ref[e][:, None, :], (L, H, masks_ref.shape[2]))

    x2d = x_ref[...].reshape(N, L)
    wt = wt_ref[...]

    lgT = jax.lax.dot_general(wt, x2d, (((1,), (1,)), ((), ())),
                              preferred_element_type=jnp.float32)
    c0 = lgT[0:1, :]
    c1 = lgT[1:2, :]
    c2 = lgT[2:3, :]

    cm = jnp.maximum(jnp.maximum(c0, c1), c2)
    e0 = jnp.exp(c0 - cm)
    e1 = jnp.exp(c1 - cm)
    e2 = jnp.exp(c2 - cm)
    s = e0 + e1 + e2
    p0 = e0 / s
    p1 = e1 / s
    p2 = e2 / s

    ge01 = p0 >= p1
    ge02 = p0 >= p2
    ge12 = p1 >= p2
    is0 = ge01 & ge02
    is1 = (~is0) & ge12
    is2 = ~(is0 | is1)
    pmax = jnp.where(is0, p0, jnp.where(is1, p1, p2))
    sec0 = (is1 & ge02) | (is2 & ge01)
    sec1 = (is0 & ge12) | (is2 & ~ge01)
    sec2 = (is0 & ~ge12) | (is1 & ~ge02)
    psec = jnp.where(sec0, p0, jnp.where(sec1, p1, p2))

    keep2 = pmax <= _TOP_P
    k2f = keep2.astype(jnp.float32)
    g0 = is0.astype(jnp.float32) + k2f * sec0.astype(jnp.float32)
    g1 = is1.astype(jnp.float32) + k2f * sec1.astype(jnp.float32)
    g2 = is2.astype(jnp.float32) + k2f * sec2.astype(jnp.float32)

    ent = -(p0 * jnp.log(p0 + _EPS)
            + p1 * jnp.log(p1 + _EPS)
            + p2 * jnp.log(p2 + _EPS))
    ent_acc[0] += jnp.sum(ent)

    imp_acc[0:1, :] += pmax
    imp_acc[1:2, :] += psec * k2f

    gpack = jnp.concatenate(
        [g0, g1, g2, jnp.zeros((5, N), jnp.float32)], axis=0)
    gcols = jnp.transpose(gpack)
    g0c = gcols[:, 0:1].reshape(1, L, H, 1)
    g1c = gcols[:, 1:2].reshape(1, L, H, 1)
    g2c = gcols[:, 2:3].reshape(1, L, H, 1)

    rows = jax.lax.broadcasted_iota(jnp.int32, (1, L, H, L), 1)
    cols = jax.lax.broadcasted_iota(jnp.int32, (1, L, H, L), 3)
    eye = (rows == cols).astype(jnp.float32)
    out_ref[...] = (g0c * mb_ref[0][None] + g1c * mb_ref[1][None]
                    + g2c * mb_ref[2][None] + eye)

    @pl.when(i == nsteps - 1)
    def _finalize():
        n_iota = jax.lax.broadcasted_iota(jnp.int32, (N, L), 0)
        l_iota = jax.lax.broadcasted_iota(jnp.int32, (N, L), 1)
        fold = ((n_iota // H) == l_iota).astype(jnp.float32)
        imp = jnp.dot(imp_acc[...], fold,
                      preferred_element_type=jnp.float32)
        n = jnp.float32(3 * L)
        mean = jnp.sum(imp) / n
        d = imp - mean
        ssq = jnp.sum(d * d) + jnp.float32(L) * mean * mean
        var = ssq / (n - 1.0)
        loss_imp = var / (mean * mean + _EPS)
        loss_dyn = ent_acc[0] / jnp.float32(3 * 512)
        loss_ref[0] = loss_imp + 0.1 * loss_dyn


@functools.partial(jax.jit, static_argnames=())
def kernel(x, masks, W_gate):
    B, H, L, _ = x.shape
    xp = jnp.transpose(x, (0, 2, 1, 3))
    masks_t = jnp.transpose(masks, (1, 0, 2))
    w_t = jnp.transpose(W_gate)

    out, loss1 = pl.pallas_call(
        _body,
        grid=(B,),
        in_specs=[
            pl.BlockSpec((1, L, H, L), lambda i: (i, 0, 0, 0)),
            pl.BlockSpec((3, L, L), lambda i: (0, 0, 0)),
            pl.BlockSpec((3, L), lambda i: (0, 0)),
        ],
        out_specs=[
            pl.BlockSpec((1, L, H, L), lambda i: (i, 0, 0, 0)),
            pl.BlockSpec(memory_space=pltpu.SMEM),
        ],
        out_shape=[
            jax.ShapeDtypeStruct((B, L, H, L), jnp.float32),
            jax.ShapeDtypeStruct((1,), jnp.float32),
        ],
        scratch_shapes=[
            pltpu.VMEM((3, L, H, L), jnp.float32),
            pltpu.VMEM((2, L * H), jnp.float32),
            pltpu.SMEM((1,), jnp.float32),
        ],
    )(xp, masks_t, w_t)

    return jnp.transpose(out, (0, 2, 1, 3)), loss1[0]


# speedup vs baseline: 6.3053x; 1.2018x over previous
import functools

import jax
import jax.numpy as jnp
from jax.experimental import pallas as pl
from jax.experimental.pallas import tpu as pltpu

_EPS = 1e-10
_TOP_P = 0.5


def _body(x_ref, masks_ref, wt_ref, out_ref, loss_ref, mb_ref, imp_acc,
          ent_acc):
    i = pl.program_id(0)
    nsteps = pl.num_programs(0)
    Kb = x_ref.shape[0]
    L = x_ref.shape[1]
    H = x_ref.shape[2]
    N = Kb * L * H

    @pl.when(i == 0)
    def _init():
        imp_acc[...] = jnp.zeros_like(imp_acc)
        ent_acc[0] = jnp.float32(0.0)
        for e in range(3):
            mb_ref[e] = jnp.broadcast_to(
                masks_ref[e][:, None, :], (L, H, masks_ref.shape[2]))

    x2d = x_ref[...].reshape(N, L)
    wt = wt_ref[...]

    lgT = jax.lax.dot_general(wt, x2d, (((1,), (1,)), ((), ())),
                              preferred_element_type=jnp.float32)
    c0 = lgT[0:1, :]
    c1 = lgT[1:2, :]
    c2 = lgT[2:3, :]

    cm = jnp.maximum(jnp.maximum(c0, c1), c2)
    e0 = jnp.exp(c0 - cm)
    e1 = jnp.exp(c1 - cm)
    e2 = jnp.exp(c2 - cm)
    s = e0 + e1 + e2
    p0 = e0 / s
    p1 = e1 / s
    p2 = e2 / s

    ge01 = p0 >= p1
    ge02 = p0 >= p2
    ge12 = p1 >= p2
    is0 = ge01 & ge02
    is1 = (~is0) & ge12
    is2 = ~(is0 | is1)
    pmax = jnp.where(is0, p0, jnp.where(is1, p1, p2))
    sec0 = (is1 & ge02) | (is2 & ge01)
    sec1 = (is0 & ge12) | (is2 & ~ge01)
    sec2 = (is0 & ~ge12) | (is1 & ~ge02)
    psec = jnp.where(sec0, p0, jnp.where(sec1, p1, p2))

    keep2 = pmax <= _TOP_P
    k2f = keep2.astype(jnp.float32)
    g0 = is0.astype(jnp.float32) + k2f * sec0.astype(jnp.float32)
    g1 = is1.astype(jnp.float32) + k2f * sec1.astype(jnp.float32)
    g2 = is2.astype(jnp.float32) + k2f * sec2.astype(jnp.float32)

    ent = -(p0 * jnp.log(p0 + _EPS)
            + p1 * jnp.log(p1 + _EPS)
            + p2 * jnp.log(p2 + _EPS))
    ent_acc[0] += jnp.sum(ent)

    imp_acc[0:1, :] += pmax
    imp_acc[1:2, :] += psec * k2f

    gpack = jnp.concatenate(
        [g0, g1, g2, jnp.zeros((5, N), jnp.float32)], axis=0)
    gcols = jnp.transpose(gpack)
    g0c = gcols[:, 0:1].reshape(Kb, L, H, 1)
    g1c = gcols[:, 1:2].reshape(Kb, L, H, 1)
    g2c = gcols[:, 2:3].reshape(Kb, L, H, 1)

    rows = jax.lax.broadcasted_iota(jnp.int32, (Kb, L, H, L), 1)
    cols = jax.lax.broadcasted_iota(jnp.int32, (Kb, L, H, L), 3)
    eye = (rows == cols).astype(jnp.float32)
    out_ref[...] = (g0c * mb_ref[0][None] + g1c * mb_ref[1][None]
                    + g2c * mb_ref[2][None] + eye)

    @pl.when(i == nsteps - 1)
    def _finalize():
        n_iota = jax.lax.broadcasted_iota(jnp.int32, (N, L), 0)
        l_iota = jax.lax.broadcasted_iota(jnp.int32, (N, L), 1)
        fold = (((n_iota // H) % L) == l_iota).astype(jnp.float32)
        imp = jnp.dot(imp_acc[...], fold,
                      preferred_element_type=jnp.float32)
        n = jnp.float32(3 * L)
        mean = jnp.sum(imp) / n
        d = imp - mean
        ssq = jnp.sum(d * d) + jnp.float32(L) * mean * mean
        var = ssq / (n - 1.0)
        loss_imp = var / (mean * mean + _EPS)
        loss_dyn = ent_acc[0] / jnp.float32(3 * 512)
        loss_ref[0] = loss_imp + 0.1 * loss_dyn


@functools.partial(jax.jit, static_argnames=())
def kernel(x, masks, W_gate):
    B, H, L, _ = x.shape
    xp = jnp.transpose(x, (0, 2, 1, 3))
    masks_t = jnp.transpose(masks, (1, 0, 2))
    w_t = jnp.transpose(W_gate)

    Kb = 2
    out, loss1 = pl.pallas_call(
        _body,
        grid=(B // Kb,),
        in_specs=[
            pl.BlockSpec((Kb, L, H, L), lambda i: (i, 0, 0, 0)),
            pl.BlockSpec((3, L, L), lambda i: (0, 0, 0)),
            pl.BlockSpec((3, L), lambda i: (0, 0)),
        ],
        out_specs=[
            pl.BlockSpec((Kb, L, H, L), lambda i: (i, 0, 0, 0)),
            pl.BlockSpec(memory_space=pltpu.SMEM),
        ],
        out_shape=[
            jax.ShapeDtypeStruct((B, L, H, L), jnp.float32),
            jax.ShapeDtypeStruct((1,), jnp.float32),
        ],
        scratch_shapes=[
            pltpu.VMEM((3, L, H, L), jnp.float32),
            pltpu.VMEM((2, Kb * L * H), jnp.float32),
            pltpu.SMEM((1,), jnp.float32),
        ],
    )(xp, masks_t, w_t)

    return jnp.transpose(out, (0, 2, 1, 3)), loss1[0]


# speedup vs baseline: 6.8370x; 1.0843x over previous
import functools

import jax
import jax.numpy as jnp
from jax.experimental import pallas as pl
from jax.experimental.pallas import tpu as pltpu

_EPS = 1e-10
_TOP_P = 0.5


def _body(x_ref, masks_ref, wt_ref, out_ref, loss_ref, mb_ref, imp_acc,
          ent_acc):
    i = pl.program_id(0)
    nsteps = pl.num_programs(0)
    Kb = x_ref.shape[0]
    L = x_ref.shape[1]
    H = x_ref.shape[2]
    N = Kb * L * H

    @pl.when(i == 0)
    def _init():
        imp_acc[...] = jnp.zeros_like(imp_acc)
        ent_acc[0] = jnp.float32(0.0)
        for e in range(3):
            mb_ref[e] = jnp.broadcast_to(
                masks_ref[e][:, None, :], (L, H, masks_ref.shape[2]))

    x2d = x_ref[...].reshape(N, L)
    wt = wt_ref[...]

    lgT = jax.lax.dot_general(wt, x2d, (((1,), (1,)), ((), ())),
                              preferred_element_type=jnp.float32)
    c0 = lgT[0:1, :]
    c1 = lgT[1:2, :]
    c2 = lgT[2:3, :]

    cm = jnp.maximum(jnp.maximum(c0, c1), c2)
    e0 = jnp.exp(c0 - cm)
    e1 = jnp.exp(c1 - cm)
    e2 = jnp.exp(c2 - cm)
    s = e0 + e1 + e2
    p0 = e0 / s
    p1 = e1 / s
    p2 = e2 / s

    ge01 = p0 >= p1
    ge02 = p0 >= p2
    ge12 = p1 >= p2
    is0 = ge01 & ge02
    is1 = (~is0) & ge12
    is2 = ~(is0 | is1)
    pmax = jnp.where(is0, p0, jnp.where(is1, p1, p2))
    sec0 = (is1 & ge02) | (is2 & ge01)
    sec1 = (is0 & ge12) | (is2 & ~ge01)
    sec2 = (is0 & ~ge12) | (is1 & ~ge02)
    psec = jnp.where(sec0, p0, jnp.where(sec1, p1, p2))

    keep2 = pmax <= _TOP_P
    k2f = keep2.astype(jnp.float32)
    g0 = is0.astype(jnp.float32) + k2f * sec0.astype(jnp.float32)
    g1 = is1.astype(jnp.float32) + k2f * sec1.astype(jnp.float32)
    g2 = is2.astype(jnp.float32) + k2f * sec2.astype(jnp.float32)

    ent = -(p0 * jnp.log(p0 + _EPS)
            + p1 * jnp.log(p1 + _EPS)
            + p2 * jnp.log(p2 + _EPS))
    ent_acc[0] += jnp.sum(ent)

    imp_acc[0:1, :] += pmax
    imp_acc[1:2, :] += psec * k2f

    gpack = jnp.concatenate(
        [g0, g1, g2, jnp.zeros((5, N), jnp.float32)], axis=0)
    gcols = jnp.transpose(gpack)
    g0c = gcols[:, 0:1].reshape(Kb, L, H, 1)
    g1c = gcols[:, 1:2].reshape(Kb, L, H, 1)
    g2c = gcols[:, 2:3].reshape(Kb, L, H, 1)

    rows = jax.lax.broadcasted_iota(jnp.int32, (Kb, L, H, L), 1)
    cols = jax.lax.broadcasted_iota(jnp.int32, (Kb, L, H, L), 3)
    eye = (rows == cols).astype(jnp.float32)
    out_ref[...] = (g0c * mb_ref[0][None] + g1c * mb_ref[1][None]
                    + g2c * mb_ref[2][None] + eye)

    @pl.when(i == nsteps - 1)
    def _finalize():
        n_iota = jax.lax.broadcasted_iota(jnp.int32, (N, L), 0)
        l_iota = jax.lax.broadcasted_iota(jnp.int32, (N, L), 1)
        fold = (((n_iota // H) % L) == l_iota).astype(jnp.float32)
        imp = jnp.dot(imp_acc[...], fold,
                      preferred_element_type=jnp.float32)
        n = jnp.float32(3 * L)
        mean = jnp.sum(imp) / n
        d = imp - mean
        ssq = jnp.sum(d * d) + jnp.float32(L) * mean * mean
        var = ssq / (n - 1.0)
        loss_imp = var / (mean * mean + _EPS)
        loss_dyn = ent_acc[0] / jnp.float32(3 * 512)
        loss_ref[0] = loss_imp + 0.1 * loss_dyn


@functools.partial(jax.jit, static_argnames=())
def kernel(x, masks, W_gate):
    B, H, L, _ = x.shape
    xp = jnp.transpose(x, (0, 2, 1, 3))
    masks_t = jnp.transpose(masks, (1, 0, 2))
    w_t = jnp.transpose(W_gate)

    Kb = 4
    out, loss1 = pl.pallas_call(
        _body,
        grid=(B // Kb,),
        in_specs=[
            pl.BlockSpec((Kb, L, H, L), lambda i: (i, 0, 0, 0)),
            pl.BlockSpec((3, L, L), lambda i: (0, 0, 0)),
            pl.BlockSpec((3, L), lambda i: (0, 0)),
        ],
        out_specs=[
            pl.BlockSpec((Kb, L, H, L), lambda i: (i, 0, 0, 0)),
            pl.BlockSpec(memory_space=pltpu.SMEM),
        ],
        out_shape=[
            jax.ShapeDtypeStruct((B, L, H, L), jnp.float32),
            jax.ShapeDtypeStruct((1,), jnp.float32),
        ],
        scratch_shapes=[
            pltpu.VMEM((3, L, H, L), jnp.float32),
            pltpu.VMEM((2, Kb * L * H), jnp.float32),
            pltpu.SMEM((1,), jnp.float32),
        ],
    )(xp, masks_t, w_t)

    return jnp.transpose(out, (0, 2, 1, 3)), loss1[0]
